# baseline (speedup 1.0000x reference)
import functools

import jax
import jax.numpy as jnp
from jax import lax
from jax.experimental import pallas as pl
from jax.experimental.pallas import tpu as pltpu
from jax.experimental.pallas import tpu_sc as plsc

N = 10000
E = 320000
D = 128
DE = 16
H = 128
L = 2

NC = 2
NS = 16
NW = NC * NS
CHUNK = 128
NCHUNK = 82
EPT = NCHUNK * CHUNK
E_PAD = NW * EPT
ROWS_PER_TILE = 632
ACC_ROWS = NS * ROWS_PER_TILE
WEP = 128
G3 = 3 * H

_sc_mesh = plsc.VectorSubcoreMesh(core_axis_name="c", subcore_axis_name="s")


@functools.partial(
    pl.kernel,
    mesh=_sc_mesh,
    out_type=jax.ShapeDtypeStruct((NC, ACC_ROWS, D), jnp.float32),
    scratch_types=[
        pltpu.VMEM((2, 1, CHUNK), jnp.int32),
        pltpu.VMEM((2, 1, CHUNK), jnp.int32),
        pltpu.VMEM_SHARED((ACC_ROWS, D), jnp.float32),
        pltpu.VMEM((CHUNK, D), jnp.float32),
        pltpu.VMEM((CHUNK, D), jnp.float32),
    ] + [pltpu.SemaphoreType.DMA] * 6,
)
def _sc_gather_segsum(cur_hbm, src_hbm, dst_hbm, zeros_hbm, out_hbm,
                      sring, dring, acc_sh, buf0, buf1,
                      g0, g1, i0, i1, j0, j1):
    bufs = (buf0, buf1)
    gsems = (g0, g1)
    isems = (i0, i1)
    jsems = (j0, j1)
    cid = lax.axis_index("c")
    sid = lax.axis_index("s")
    w = cid * NS + sid
    pltpu.sync_copy(zeros_hbm, acc_sh.at[pl.ds(sid * ROWS_PER_TILE, ROWS_PER_TILE)])
    plsc.subcore_barrier()

    def idx_load(c, s):
        pltpu.async_copy(src_hbm.at[w * NCHUNK + c], sring.at[s], isems[s])
        pltpu.async_copy(dst_hbm.at[w * NCHUNK + c], dring.at[s], jsems[s])

    def swait(s):
        pltpu.make_async_copy(src_hbm.at[0], sring.at[s], isems[s]).wait()

    def jwait(s):
        pltpu.make_async_copy(dst_hbm.at[0], dring.at[s], jsems[s]).wait()

    def row_start(c, s):
        pltpu.async_copy(cur_hbm.at[sring.at[s, 0]], bufs[s], gsems[s])

    def row_wait(c, s):
        pltpu.make_async_copy(cur_hbm.at[sring.at[s, 0]], bufs[s], gsems[s]).wait()

    idx_load(0, 0)
    idx_load(1, 1)
    swait(0)
    row_start(0, 0)

    def outer(cc2, carry):
        for s in (0, 1):
            c = cc2 * 2 + s
            row_wait(c, s)

            @pl.when(c + 1 < NCHUNK)
            def _():
                swait(1 - s)
                row_start(c + 1, 1 - s)
            jwait(s)
            pltpu.sync_copy(bufs[s], acc_sh.at[dring.at[s, 0]], add=True)

            @pl.when(c + 2 < NCHUNK)
            def _():
                idx_load(c + 2, s)
        return carry

    lax.fori_loop(0, NCHUNK // 2, outer, None)
    plsc.subcore_barrier()
    pltpu.sync_copy(
        acc_sh.at[pl.ds(sid * ROWS_PER_TILE, ROWS_PER_TILE)],
        out_hbm.at[cid, pl.ds(sid * ROWS_PER_TILE, ROWS_PER_TILE)],
    )


@functools.partial(
    pl.kernel,
    mesh=_sc_mesh,
    out_type=jax.ShapeDtypeStruct((NC, ACC_ROWS, WEP), jnp.float32),
    scratch_types=[
        pltpu.VMEM((2, 1, CHUNK), jnp.int32),
        pltpu.VMEM_SHARED((ACC_ROWS, WEP), jnp.float32),
        pltpu.VMEM((CHUNK, WEP), jnp.float32),
        pltpu.VMEM((CHUNK, WEP), jnp.float32),
    ] + [pltpu.SemaphoreType.DMA] * 4,
)
def _sc_edge_segsum(payload_hbm, dst_hbm, zeros_hbm, out_hbm,
                    dring, acc_sh, buf0, buf1, g0, g1, j0, j1):
    bufs = (buf0, buf1)
    gsems = (g0, g1)
    jsems = (j0, j1)
    cid = lax.axis_index("c")
    sid = lax.axis_index("s")
    w = cid * NS + sid
    pltpu.sync_copy(zeros_hbm, acc_sh.at[pl.ds(sid * ROWS_PER_TILE, ROWS_PER_TILE)])
    plsc.subcore_barrier()

    def idx_load(c, s):
        pltpu.async_copy(dst_hbm.at[w * NCHUNK + c], dring.at[s], jsems[s])

    def jwait(s):
        pltpu.make_async_copy(dst_hbm.at[0], dring.at[s], jsems[s]).wait()

    def row_start(c, s):
        pltpu.async_copy(payload_hbm.at[pl.ds(w * EPT + c * CHUNK, CHUNK)],
                         bufs[s], gsems[s])

    def row_wait(c, s):
        pltpu.make_async_copy(payload_hbm.at[pl.ds(0, CHUNK)],
                              bufs[s], gsems[s]).wait()

    idx_load(0, 0)
    idx_load(1, 1)
    row_start(0, 0)

    def outer(cc2, carry):
        for s in (0, 1):
            c = cc2 * 2 + s
            row_wait(c, s)

            @pl.when(c + 1 < NCHUNK)
            def _():
                row_start(c + 1, 1 - s)
            jwait(s)
            pltpu.sync_copy(bufs[s], acc_sh.at[dring.at[s, 0]], add=True)

            @pl.when(c + 2 < NCHUNK)
            def _():
                idx_load(c + 2, s)
        return carry

    lax.fori_loop(0, NCHUNK // 2, outer, None)
    plsc.subcore_barrier()
    pltpu.sync_copy(
        acc_sh.at[pl.ds(sid * ROWS_PER_TILE, ROWS_PER_TILE)],
        out_hbm.at[cid, pl.ds(sid * ROWS_PER_TILE, ROWS_PER_TILE)],
    )


_BR = 2528


def _tc_dense_body(g0_ref, g1_ref, e0_ref, e1_ref, wn_ref, we_ref, out_ref):
    g = g0_ref[...] + g1_ref[...]
    e = e0_ref[...] + e1_ref[...]
    agg = lax.dot_general(g, wn_ref[...], (((1,), (0,)), ((), ())),
                          preferred_element_type=jnp.float32)
    agg += lax.dot_general(e, we_ref[...], (((1,), (0,)), ((), ())),
                           preferred_element_type=jnp.float32)
    agg = jnp.maximum(agg, 0.0)
    i = jax.nn.sigmoid(agg[:, :H])
    ct = jnp.tanh(agg[:, H:2 * H])
    o = jax.nn.sigmoid(agg[:, 2 * H:])
    out_ref[...] = o * jnp.tanh(i * ct)


def _tc_dense(g0, g1, e0, e1, wn3, we3p):
    return pl.pallas_call(
        _tc_dense_body,
        grid=(ACC_ROWS // _BR,),
        in_specs=[
            pl.BlockSpec((_BR, D), lambda i: (i, 0)),
            pl.BlockSpec((_BR, D), lambda i: (i, 0)),
            pl.BlockSpec((_BR, WEP), lambda i: (i, 0)),
            pl.BlockSpec((_BR, WEP), lambda i: (i, 0)),
            pl.BlockSpec((D, G3), lambda i: (0, 0)),
            pl.BlockSpec((WEP, G3), lambda i: (0, 0)),
        ],
        out_specs=pl.BlockSpec((_BR, D), lambda i: (i, 0)),
        out_shape=jax.ShapeDtypeStruct((ACC_ROWS, D), jnp.float32),
    )(g0, g1, e0, e1, wn3, we3p)


_GATES = (0, 2, 3)


def kernel(x, edge_index, edge_attr, Wn, bn, We, be):
    src = edge_index[0]
    dst = edge_index[1]
    pad = E_PAD - E
    srcp = jnp.concatenate([src, jnp.zeros((pad,), jnp.int32)])
    dstp = jnp.concatenate([dst, jnp.full((pad,), N, jnp.int32)])
    src3 = srcp.reshape(NW * NCHUNK, 1, CHUNK)
    dst3 = dstp.reshape(NW * NCHUNK, 1, CHUNK)
    payload = jnp.concatenate(
        [edge_attr,
         jnp.ones((E, 1), jnp.float32),
         jnp.zeros((E, WEP - DE - 1), jnp.float32)], axis=1)
    payload = jnp.concatenate(
        [payload, jnp.zeros((pad, WEP), jnp.float32)], axis=0)
    zeros_d = jnp.zeros((ROWS_PER_TILE, D), jnp.float32)
    zeros_e = jnp.zeros((ROWS_PER_TILE, WEP), jnp.float32)

    ea = _sc_edge_segsum(payload, dst3, zeros_e)

    def mk_w(l):
        wn3 = jnp.concatenate([Wn[l, g, :D, :] for g in _GATES], axis=1)
        wep = jnp.concatenate([We[l, g] for g in _GATES], axis=1)
        brow = jnp.concatenate([bn[l, g] + be[l, g] for g in _GATES])[None, :]
        we3p = jnp.concatenate(
            [wep, brow, jnp.zeros((WEP - DE - 1, G3), jnp.float32)], axis=0)
        return wn3, we3p

    cur = x
    for l in range(L):
        g = _sc_gather_segsum(cur, src3, dst3, zeros_d)
        wn3, we3p = mk_w(l)
        cur = _tc_dense(g[0], g[1], ea[0], ea[1], wn3, we3p)
    return cur[:N]


# speedup vs baseline: 1.6115x; 1.6115x over previous
import functools

import jax
import jax.numpy as jnp
from jax import lax
from jax.experimental import pallas as pl
from jax.experimental.pallas import tpu as pltpu
from jax.experimental.pallas import tpu_sc as plsc

N = 10000
E = 320000
D = 128
DE = 16
H = 128
L = 2

NC = 2
NS = 16
NW = NC * NS
CHUNK = 128
NCHUNK = 80
EPT = NCHUNK * CHUNK
E_PAD = NW * EPT
ROWS_PER_TILE = 632
ACC_ROWS = NS * ROWS_PER_TILE
WEP = 128
G3 = 3 * H

_sc_mesh = plsc.VectorSubcoreMesh(core_axis_name="c", subcore_axis_name="s")


@functools.partial(
    pl.kernel,
    mesh=_sc_mesh,
    out_type=jax.ShapeDtypeStruct((NC, ACC_ROWS, D), jnp.float32),
    scratch_types=[
        pltpu.VMEM((NCHUNK, CHUNK), jnp.int32),
        pltpu.VMEM((2, 1, CHUNK), jnp.int32),
        pltpu.VMEM_SHARED((ACC_ROWS, D), jnp.float32),
        pltpu.VMEM((CHUNK, D), jnp.float32),
        pltpu.VMEM((CHUNK, D), jnp.float32),
        pltpu.SemaphoreType.DMA,
        pltpu.SemaphoreType.DMA,
        pltpu.SemaphoreType.DMA,
        pltpu.SemaphoreType.DMA,
    ],
)
def _sc_gather_segsum(cur_hbm, src_hbm, dst_hbm, zeros_hbm, out_hbm,
                      src_v, dring, acc_sh, buf0, buf1, g0, g1, j0, j1):
    bufs = (buf0, buf1)
    gsems = (g0, g1)
    jsems = (j0, j1)
    cid = lax.axis_index("c")
    sid = lax.axis_index("s")
    w = cid * NS + sid
    pltpu.sync_copy(zeros_hbm, acc_sh.at[pl.ds(sid * ROWS_PER_TILE, ROWS_PER_TILE)])
    pltpu.sync_copy(src_hbm.at[w], src_v)
    plsc.subcore_barrier()

    def jload(c, s):
        pltpu.async_copy(dst_hbm.at[w * NCHUNK + c], dring.at[s], jsems[s])

    def jwait(s):
        pltpu.make_async_copy(dst_hbm.at[0], dring.at[s], jsems[s]).wait()

    def row_start(c, s):
        pltpu.async_copy(cur_hbm.at[src_v.at[c]], bufs[s], gsems[s])

    def row_wait(c, s):
        pltpu.make_async_copy(cur_hbm.at[src_v.at[c]], bufs[s], gsems[s]).wait()

    jload(0, 0)
    jload(1, 1)
    row_start(0, 0)

    def outer(cc2, carry):
        for s in (0, 1):
            c = cc2 * 2 + s
            row_wait(c, s)

            @pl.when(c + 1 < NCHUNK)
            def _():
                row_start(c + 1, 1 - s)
            jwait(s)
            pltpu.sync_copy(bufs[s], acc_sh.at[dring.at[s, 0]], add=True)

            @pl.when(c + 2 < NCHUNK)
            def _():
                jload(c + 2, s)
        return carry

    lax.fori_loop(0, NCHUNK // 2, outer, None)
    plsc.subcore_barrier()
    pltpu.sync_copy(
        acc_sh.at[pl.ds(sid * ROWS_PER_TILE, ROWS_PER_TILE)],
        out_hbm.at[cid, pl.ds(sid * ROWS_PER_TILE, ROWS_PER_TILE)],
    )


@functools.partial(
    pl.kernel,
    mesh=_sc_mesh,
    out_type=jax.ShapeDtypeStruct((NC, ACC_ROWS, WEP), jnp.float32),
    scratch_types=[
        pltpu.VMEM((2, 1, CHUNK), jnp.int32),
        pltpu.VMEM_SHARED((ACC_ROWS, WEP), jnp.float32),
        pltpu.VMEM((CHUNK, WEP), jnp.float32),
        pltpu.VMEM((CHUNK, WEP), jnp.float32),
        pltpu.SemaphoreType.DMA,
        pltpu.SemaphoreType.DMA,
        pltpu.SemaphoreType.DMA,
        pltpu.SemaphoreType.DMA,
    ],
)
def _sc_edge_segsum(payload_hbm, dst_hbm, zeros_hbm, out_hbm,
                    dring, acc_sh, buf0, buf1, g0, g1, j0, j1):
    bufs = (buf0, buf1)
    gsems = (g0, g1)
    jsems = (j0, j1)
    cid = lax.axis_index("c")
    sid = lax.axis_index("s")
    w = cid * NS + sid
    pltpu.sync_copy(zeros_hbm, acc_sh.at[pl.ds(sid * ROWS_PER_TILE, ROWS_PER_TILE)])
    plsc.subcore_barrier()

    def jload(c, s):
        pltpu.async_copy(dst_hbm.at[w * NCHUNK + c], dring.at[s], jsems[s])

    def jwait(s):
        pltpu.make_async_copy(dst_hbm.at[0], dring.at[s], jsems[s]).wait()

    def row_start(c, s):
        pltpu.async_copy(payload_hbm.at[pl.ds(w * EPT + c * CHUNK, CHUNK)],
                         bufs[s], gsems[s])

    def row_wait(c, s):
        pltpu.make_async_copy(payload_hbm.at[pl.ds(0, CHUNK)],
                              bufs[s], gsems[s]).wait()

    jload(0, 0)
    jload(1, 1)
    row_start(0, 0)

    def outer(cc2, carry):
        for s in (0, 1):
            c = cc2 * 2 + s
            row_wait(c, s)

            @pl.when(c + 1 < NCHUNK)
            def _():
                row_start(c + 1, 1 - s)
            jwait(s)
            pltpu.sync_copy(bufs[s], acc_sh.at[dring.at[s, 0]], add=True)

            @pl.when(c + 2 < NCHUNK)
            def _():
                jload(c + 2, s)
        return carry

    lax.fori_loop(0, NCHUNK // 2, outer, None)
    plsc.subcore_barrier()
    pltpu.sync_copy(
        acc_sh.at[pl.ds(sid * ROWS_PER_TILE, ROWS_PER_TILE)],
        out_hbm.at[cid, pl.ds(sid * ROWS_PER_TILE, ROWS_PER_TILE)],
    )


_BR = 2528


def _tc_dense_body(g0_ref, g1_ref, e0_ref, e1_ref, wn_ref, we_ref, out_ref):
    g = g0_ref[...] + g1_ref[...]
    e = e0_ref[...] + e1_ref[...]
    agg = lax.dot_general(g, wn_ref[...], (((1,), (0,)), ((), ())),
                          preferred_element_type=jnp.float32)
    agg += lax.dot_general(e, we_ref[...], (((1,), (0,)), ((), ())),
                           preferred_element_type=jnp.float32)
    agg = jnp.maximum(agg, 0.0)
    i = jax.nn.sigmoid(agg[:, :H])
    ct = jnp.tanh(agg[:, H:2 * H])
    o = jax.nn.sigmoid(agg[:, 2 * H:])
    out_ref[...] = o * jnp.tanh(i * ct)


def _tc_dense(g0, g1, e0, e1, wn3, we3p):
    return pl.pallas_call(
        _tc_dense_body,
        grid=(ACC_ROWS // _BR,),
        in_specs=[
            pl.BlockSpec((_BR, D), lambda i: (i, 0)),
            pl.BlockSpec((_BR, D), lambda i: (i, 0)),
            pl.BlockSpec((_BR, WEP), lambda i: (i, 0)),
            pl.BlockSpec((_BR, WEP), lambda i: (i, 0)),
            pl.BlockSpec((D, G3), lambda i: (0, 0)),
            pl.BlockSpec((WEP, G3), lambda i: (0, 0)),
        ],
        out_specs=pl.BlockSpec((_BR, D), lambda i: (i, 0)),
        out_shape=jax.ShapeDtypeStruct((ACC_ROWS, D), jnp.float32),
    )(g0, g1, e0, e1, wn3, we3p)


_GATES = (0, 2, 3)


def kernel(x, edge_index, edge_attr, Wn, bn, We, be):
    src = edge_index[0]
    dst = edge_index[1]
    pad = E_PAD - E
    srcp = jnp.concatenate([src, jnp.zeros((pad,), jnp.int32)])
    dstp = jnp.concatenate([dst, jnp.full((pad,), N, jnp.int32)])
    src3 = srcp.reshape(NW, NCHUNK, CHUNK)
    dst3 = dstp.reshape(NW * NCHUNK, 1, CHUNK)
    payload = jnp.concatenate(
        [edge_attr,
         jnp.ones((E, 1), jnp.float32),
         jnp.zeros((E, WEP - DE - 1), jnp.float32)], axis=1)
    payload = jnp.concatenate(
        [payload, jnp.zeros((pad, WEP), jnp.float32)], axis=0)
    zeros_d = jnp.zeros((ROWS_PER_TILE, D), jnp.float32)
    zeros_e = jnp.zeros((ROWS_PER_TILE, WEP), jnp.float32)

    ea = _sc_edge_segsum(payload, dst3, zeros_e)

    def mk_w(l):
        wn3 = jnp.concatenate([Wn[l, g, :D, :] for g in _GATES], axis=1)
        wep = jnp.concatenate([We[l, g] for g in _GATES], axis=1)
        brow = jnp.concatenate([bn[l, g] + be[l, g] for g in _GATES])[None, :]
        we3p = jnp.concatenate(
            [wep, brow, jnp.zeros((WEP - DE - 1, G3), jnp.float32)], axis=0)
        return wn3, we3p

    cur = x
    for l in range(L):
        g = _sc_gather_segsum(cur, src3, dst3, zeros_d)
        wn3, we3p = mk_w(l)
        cur = _tc_dense(g[0], g[1], ea[0], ea[1], wn3, we3p)
    return cur[:N]


# speedup vs baseline: 1.6125x; 1.0006x over previous
import functools

import jax
import jax.numpy as jnp
from jax import lax
from jax.experimental import pallas as pl
from jax.experimental.pallas import tpu as pltpu
from jax.experimental.pallas import tpu_sc as plsc

N = 10000
E = 320000
D = 128
DE = 16
H = 128
L = 2

NC = 2
NS = 16
NW = NC * NS
CHUNK = 128
NCHUNK = 80
EPT = NCHUNK * CHUNK
E_PAD = NW * EPT
ROWS_PER_TILE = 632
ACC_ROWS = NS * ROWS_PER_TILE
WEP = 128
G3 = 3 * H

_sc_mesh = plsc.VectorSubcoreMesh(core_axis_name="c", subcore_axis_name="s")


@functools.partial(
    pl.kernel,
    mesh=_sc_mesh,
    out_type=jax.ShapeDtypeStruct((NC, ACC_ROWS, D), jnp.float32),
    scratch_types=[
        pltpu.VMEM((NCHUNK, CHUNK), jnp.int32),
        pltpu.VMEM((2, 1, CHUNK), jnp.int32),
        pltpu.VMEM_SHARED((ACC_ROWS, D), jnp.float32),
        pltpu.VMEM((CHUNK, D), jnp.float32),
        pltpu.VMEM((CHUNK, D), jnp.float32),
        pltpu.SemaphoreType.DMA,
        pltpu.SemaphoreType.DMA,
        pltpu.SemaphoreType.DMA,
        pltpu.SemaphoreType.DMA,
    ],
)
def _sc_gather_segsum(cur_hbm, src_hbm, dst_hbm, zeros_hbm, out_hbm,
                      src_v, dring, acc_sh, buf0, buf1, g0, g1, j0, j1):
    bufs = (buf0, buf1)
    gsems = (g0, g1)
    jsems = (j0, j1)
    cid = lax.axis_index("c")
    sid = lax.axis_index("s")
    w = cid * NS + sid
    pltpu.sync_copy(zeros_hbm, acc_sh.at[pl.ds(sid * ROWS_PER_TILE, ROWS_PER_TILE)])
    pltpu.sync_copy(src_hbm.at[w], src_v)
    plsc.subcore_barrier()

    def jload(c, s):
        pltpu.async_copy(dst_hbm.at[w * NCHUNK + c], dring.at[s], jsems[s])

    def jwait(s):
        pltpu.make_async_copy(dst_hbm.at[0], dring.at[s], jsems[s]).wait()

    def row_start(c, s):
        pltpu.async_copy(cur_hbm.at[src_v.at[c]], bufs[s], gsems[s])

    def row_wait(c, s):
        pltpu.make_async_copy(cur_hbm.at[src_v.at[c]], bufs[s], gsems[s]).wait()

    jload(0, 0)
    jload(1, 1)
    row_start(0, 0)

    def outer(cc2, carry):
        for s in (0, 1):
            c = cc2 * 2 + s
            row_wait(c, s)

            @pl.when(c + 1 < NCHUNK)
            def _():
                row_start(c + 1, 1 - s)
            jwait(s)
            pltpu.sync_copy(bufs[s], acc_sh.at[dring.at[s, 0]], add=True)

            @pl.when(c + 2 < NCHUNK)
            def _():
                jload(c + 2, s)
        return carry

    lax.fori_loop(0, NCHUNK // 2, outer, None)
    plsc.subcore_barrier()
    pltpu.sync_copy(
        acc_sh.at[pl.ds(sid * ROWS_PER_TILE, ROWS_PER_TILE)],
        out_hbm.at[cid, pl.ds(sid * ROWS_PER_TILE, ROWS_PER_TILE)],
    )


@functools.partial(
    pl.kernel,
    mesh=_sc_mesh,
    out_type=jax.ShapeDtypeStruct((NC, ACC_ROWS, WEP), jnp.float32),
    scratch_types=[
        pltpu.VMEM((2, 1, CHUNK), jnp.int32),
        pltpu.VMEM_SHARED((ACC_ROWS, WEP), jnp.float32),
        pltpu.VMEM((CHUNK, WEP), jnp.float32),
        pltpu.VMEM((CHUNK, WEP), jnp.float32),
        pltpu.SemaphoreType.DMA,
        pltpu.SemaphoreType.DMA,
        pltpu.SemaphoreType.DMA,
        pltpu.SemaphoreType.DMA,
    ],
)
def _sc_edge_segsum(payload_hbm, dst_hbm, zeros_hbm, out_hbm,
                    dring, acc_sh, buf0, buf1, g0, g1, j0, j1):
    bufs = (buf0, buf1)
    gsems = (g0, g1)
    jsems = (j0, j1)
    cid = lax.axis_index("c")
    sid = lax.axis_index("s")
    w = cid * NS + sid
    pltpu.sync_copy(zeros_hbm, acc_sh.at[pl.ds(sid * ROWS_PER_TILE, ROWS_PER_TILE)])
    plsc.subcore_barrier()

    def jload(c, s):
        pltpu.async_copy(dst_hbm.at[w * NCHUNK + c], dring.at[s], jsems[s])

    def jwait(s):
        pltpu.make_async_copy(dst_hbm.at[0], dring.at[s], jsems[s]).wait()

    def row_start(c, s):
        pltpu.async_copy(payload_hbm.at[pl.ds(w * EPT + c * CHUNK, CHUNK)],
                         bufs[s], gsems[s])

    def row_wait(c, s):
        pltpu.make_async_copy(payload_hbm.at[pl.ds(0, CHUNK)],
                              bufs[s], gsems[s]).wait()

    jload(0, 0)
    jload(1, 1)
    row_start(0, 0)

    def outer(cc2, carry):
        for s in (0, 1):
            c = cc2 * 2 + s
            row_wait(c, s)

            @pl.when(c + 1 < NCHUNK)
            def _():
                row_start(c + 1, 1 - s)
            jwait(s)
            pltpu.sync_copy(bufs[s], acc_sh.at[dring.at[s, 0]], add=True)

            @pl.when(c + 2 < NCHUNK)
            def _():
                jload(c + 2, s)
        return carry

    lax.fori_loop(0, NCHUNK // 2, outer, None)
    plsc.subcore_barrier()
    pltpu.sync_copy(
        acc_sh.at[pl.ds(sid * ROWS_PER_TILE, ROWS_PER_TILE)],
        out_hbm.at[cid, pl.ds(sid * ROWS_PER_TILE, ROWS_PER_TILE)],
    )


_BR = 2528


def _tc_dense_body(g0_ref, g1_ref, e0_ref, e1_ref, wn_ref, we_ref, out_ref):
    g = g0_ref[...] + g1_ref[...]
    e = e0_ref[...] + e1_ref[...]
    agg = lax.dot_general(g, wn_ref[...], (((1,), (0,)), ((), ())),
                          preferred_element_type=jnp.float32)
    agg += lax.dot_general(e, we_ref[...], (((1,), (0,)), ((), ())),
                           preferred_element_type=jnp.float32)
    agg = jnp.maximum(agg, 0.0)
    i = jax.nn.sigmoid(agg[:, :H])
    ct = jnp.tanh(agg[:, H:2 * H])
    o = jax.nn.sigmoid(agg[:, 2 * H:])
    out_ref[...] = (o * jnp.tanh(i * ct)).astype(out_ref.dtype)


def _tc_dense(g0, g1, e0, e1, wn3, we3p, out_dtype):
    return pl.pallas_call(
        _tc_dense_body,
        grid=(ACC_ROWS // _BR,),
        in_specs=[
            pl.BlockSpec((_BR, D), lambda i: (i, 0)),
            pl.BlockSpec((_BR, D), lambda i: (i, 0)),
            pl.BlockSpec((_BR, WEP), lambda i: (i, 0)),
            pl.BlockSpec((_BR, WEP), lambda i: (i, 0)),
            pl.BlockSpec((D, G3), lambda i: (0, 0)),
            pl.BlockSpec((WEP, G3), lambda i: (0, 0)),
        ],
        out_specs=pl.BlockSpec((_BR, D), lambda i: (i, 0)),
        out_shape=jax.ShapeDtypeStruct((ACC_ROWS, D), out_dtype),
    )(g0, g1, e0, e1, wn3, we3p)


_GATES = (0, 2, 3)


def kernel(x, edge_index, edge_attr, Wn, bn, We, be):
    src = edge_index[0]
    dst = edge_index[1]
    pad = E_PAD - E
    srcp = jnp.concatenate([src, jnp.zeros((pad,), jnp.int32)])
    dstp = jnp.concatenate([dst, jnp.full((pad,), N, jnp.int32)])
    src3 = srcp.reshape(NW, NCHUNK, CHUNK)
    dst3 = dstp.reshape(NW * NCHUNK, 1, CHUNK)
    payload = jnp.concatenate(
        [edge_attr,
         jnp.ones((E, 1), jnp.float32),
         jnp.zeros((E, WEP - DE - 1), jnp.float32)], axis=1)
    payload = jnp.concatenate(
        [payload, jnp.zeros((pad, WEP), jnp.float32)], axis=0)
    zeros_d = jnp.zeros((ROWS_PER_TILE, D), jnp.float32)
    zeros_e = jnp.zeros((ROWS_PER_TILE, WEP), jnp.float32)

    ea = _sc_edge_segsum(payload, dst3, zeros_e)

    def mk_w(l):
        wn3 = jnp.concatenate([Wn[l, g, :D, :] for g in _GATES], axis=1)
        wep = jnp.concatenate([We[l, g] for g in _GATES], axis=1)
        brow = jnp.concatenate([bn[l, g] + be[l, g] for g in _GATES])[None, :]
        we3p = jnp.concatenate(
            [wep, brow, jnp.zeros((WEP - DE - 1, G3), jnp.float32)], axis=0)
        return wn3, we3p

    cur = x
    for l in range(L):
        g = _sc_gather_segsum(cur, src3, dst3, zeros_d)
        wn3, we3p = mk_w(l)
        cur = _tc_dense(g[0], g[1], ea[0], ea[1], wn3, we3p, jnp.float32)
    return cur[:N]
